# baseline (speedup 1.0000x reference)
import functools

import jax
import jax.numpy as jnp
from jax import lax
from jax.experimental import pallas as pl
from jax.experimental.pallas import tpu as pltpu
from jax.experimental.pallas import tpu_sc as plsc

E = 8
D = 768
F = 768
S = 768
T = 2048

BLK = 512
G_MOE = 16
P_MOE = G_MOE * BLK

NC, NS = 2, 16
NW = NC * NS
CHUNK = T // NW
SUB = 16
LANES = 16
GW = 128


def _router_body(logits_ref, pos0_ref, pos1_ref, g0_ref, g1_ref,
                 dma_ref, act_ref, wa_ref, wb_ref, par_ref):
    logits = logits_ref[...]
    col = lax.broadcasted_iota(jnp.int32, (T, E), 1)
    l0 = jnp.max(logits, axis=1, keepdims=True)
    i0 = jnp.min(jnp.where(logits == l0, col, E), axis=1, keepdims=True)
    masked = jnp.where(col == i0, -jnp.inf, logits)
    l1 = jnp.max(masked, axis=1, keepdims=True)
    i1 = jnp.min(jnp.where(masked == l1, col, E), axis=1, keepdims=True)
    ed = jnp.exp(l1 - l0)
    g0 = 1.0 / (1.0 + ed)
    g1 = ed / (1.0 + ed)
    g0_ref[...] = jnp.broadcast_to(g0, (T, GW))
    g1_ref[...] = jnp.broadcast_to(g1, (T, GW))

    m = ((col == i0) | (col == i1)).astype(jnp.float32)
    c = m
    sh = 1
    while sh < T:
        c = c + jnp.concatenate(
            [jnp.zeros((sh, E), jnp.float32), c[: T - sh]], axis=0)
        sh *= 2
    counts = c[T - 1: T, :]
    padded = jnp.ceil(counts * (1.0 / BLK)) * BLK
    off = padded
    sh = 1
    while sh < E:
        off = off + jnp.concatenate(
            [jnp.zeros((1, sh), jnp.float32), off[:, : E - sh]], axis=1)
        sh *= 2
    off = off - padded

    pos = off + c - 1.0
    pos0 = jnp.sum(jnp.where(col == i0, pos, 0.0), axis=1, keepdims=True)
    pos1 = jnp.sum(jnp.where(col == i1, pos, 0.0), axis=1, keepdims=True)
    pos0_ref[...] = pos0.astype(jnp.int32).reshape(16, 128)
    pos1_ref[...] = pos1.astype(jnp.int32).reshape(16, 128)

    nbb = off * (1.0 / BLK)
    used = jnp.sum(padded) * (1.0 / BLK)
    e_last = jnp.max(jnp.where(counts > 0.0,
                               col[:1, :].astype(jnp.float32), 0.0))
    gidx = lax.broadcasted_iota(jnp.int32, (G_MOE, E), 0).astype(jnp.float32)
    nbb_b = jnp.broadcast_to(nbb, (G_MOE, E))
    exp_g = jnp.sum((nbb_b <= gidx).astype(jnp.float32), axis=1,
                    keepdims=True) - 1.0
    gcol = lax.broadcasted_iota(jnp.int32, (G_MOE, 1), 0).astype(jnp.float32)
    active = gcol < used
    dma_ref[...] = jnp.where(active, exp_g, e_last).astype(jnp.int32)
    act_ref[...] = active.astype(jnp.int32)

    present = (counts > 0.0).astype(jnp.float32)
    pres_b = jnp.broadcast_to(present, (G_MOE, E))
    ecol = jnp.broadcast_to(col[:1, :].astype(jnp.float32), (G_MOE, E))
    rank = jnp.sum(jnp.where((pres_b > 0) & (nbb_b <= gidx), 1.0, 0.0),
                   axis=1, keepdims=True) - 1.0
    nxt = jnp.min(jnp.where((pres_b > 0) & (nbb_b > gidx), ecol, 1e9),
                  axis=1, keepdims=True)
    cur = exp_g
    nxt = jnp.where(nxt > 8.0, cur, nxt)
    parity = rank - 2.0 * jnp.floor(rank * 0.5)
    a_idx = jnp.where(parity < 0.5, cur, nxt)
    b_idx = jnp.where(parity < 0.5, nxt, cur)
    a_idx = jnp.where(active, a_idx, e_last)
    b_idx = jnp.where(active, b_idx, e_last)
    wa_ref[...] = a_idx.astype(jnp.int32)
    wb_ref[...] = b_idx.astype(jnp.int32)
    par_ref[...] = parity.astype(jnp.int32)


def _router_call(logits):
    return pl.pallas_call(
        _router_body,
        out_shape=(
            jax.ShapeDtypeStruct((16, 128), jnp.int32),
            jax.ShapeDtypeStruct((16, 128), jnp.int32),
            jax.ShapeDtypeStruct((T, GW), jnp.float32),
            jax.ShapeDtypeStruct((T, GW), jnp.float32),
            jax.ShapeDtypeStruct((G_MOE, 1), jnp.int32),
            jax.ShapeDtypeStruct((G_MOE, 1), jnp.int32),
            jax.ShapeDtypeStruct((G_MOE, 1), jnp.int32),
            jax.ShapeDtypeStruct((G_MOE, 1), jnp.int32),
            jax.ShapeDtypeStruct((G_MOE, 1), jnp.int32),
        ),
    )(logits)


@functools.cache
def _make_scatter():
    mesh = plsc.VectorSubcoreMesh(core_axis_name="c", subcore_axis_name="s")

    @functools.partial(
        pl.kernel,
        out_type=(
            jax.ShapeDtypeStruct((P_MOE, D), jnp.float32),
            jax.ShapeDtypeStruct((P_MOE, GW), jnp.float32),
        ),
        mesh=mesh,
        scratch_types=[
            pltpu.VMEM((CHUNK,), jnp.int32),
            pltpu.VMEM((CHUNK,), jnp.int32),
            pltpu.VMEM((CHUNK, D), jnp.float32),
            pltpu.VMEM((CHUNK, GW), jnp.float32),
            pltpu.VMEM((CHUNK, GW), jnp.float32),
            pltpu.SemaphoreType.DMA,
        ],
    )
    def scatter_k(x_hbm, pos0_hbm, pos1_hbm, g0_hbm, g1_hbm, xs_hbm, gs_hbm,
                  idx0_v, idx1_v, rows_v, gr0_v, gr1_v, sem):
        wid = lax.axis_index("s") * NC + lax.axis_index("c")
        base = wid * CHUNK
        pltpu.sync_copy(x_hbm.at[pl.ds(base, CHUNK)], rows_v)
        pltpu.sync_copy(pos0_hbm.at[pl.ds(base, CHUNK)], idx0_v)
        pltpu.sync_copy(pos1_hbm.at[pl.ds(base, CHUNK)], idx1_v)
        pltpu.sync_copy(g0_hbm.at[pl.ds(base, CHUNK)], gr0_v)
        pltpu.sync_copy(g1_hbm.at[pl.ds(base, CHUNK)], gr1_v)
        c0 = pltpu.async_copy(rows_v, xs_hbm.at[idx0_v], sem)
        c1 = pltpu.async_copy(rows_v, xs_hbm.at[idx1_v], sem)
        c2 = pltpu.async_copy(gr0_v, gs_hbm.at[idx0_v], sem)
        c3 = pltpu.async_copy(gr1_v, gs_hbm.at[idx1_v], sem)
        c0.wait()
        c1.wait()
        c2.wait()
        c3.wait()

    return scatter_k


def _gemm_body(dma_ref, act_ref, par_ref, xs_ref, gs_ref, wgu_a_ref,
               wgu_b_ref, wd_ref, y_ref, wgu_bf, wd_bf):
    g = pl.program_id(0)
    prev = dma_ref[jnp.maximum(g - 1, 0), 0]
    changed = jnp.logical_or(g == 0, dma_ref[g, 0] != prev)
    use_a = par_ref[g, 0] == 0

    @pl.when(changed & use_a)
    def _():
        wgu_bf[...] = wgu_a_ref[0].astype(jnp.bfloat16)
        wd_bf[...] = wd_ref[0].astype(jnp.bfloat16)

    @pl.when(changed & jnp.logical_not(use_a))
    def _():
        wgu_bf[...] = wgu_b_ref[0].astype(jnp.bfloat16)
        wd_bf[...] = wd_ref[0].astype(jnp.bfloat16)

    @pl.when(act_ref[g, 0] > 0)
    def _():
        x = xs_ref[...].astype(jnp.bfloat16)
        h = jnp.dot(x, wgu_bf[...], preferred_element_type=jnp.float32)
        gate = h[:, :F]
        up = h[:, F:]
        act = (gate * (1.0 / (1.0 + jnp.exp(-gate))) * up).astype(jnp.bfloat16)
        y = jnp.dot(act, wd_bf[...], preferred_element_type=jnp.float32)
        y_ref[...] = y * gs_ref[:, :1]


def _gemm_call(dma, act, wa, wb, par, xs, gs, w_gate_up, w_down):
    grid_spec = pltpu.PrefetchScalarGridSpec(
        num_scalar_prefetch=5,
        grid=(G_MOE,),
        in_specs=[
            pl.BlockSpec((BLK, D), lambda g, d, a, wa, wb, p: (g, 0)),
            pl.BlockSpec((BLK, GW), lambda g, d, a, wa, wb, p: (g, 0)),
            pl.BlockSpec((1, D, 2 * F),
                         lambda g, d, a, wa, wb, p: (wa[g, 0], 0, 0)),
            pl.BlockSpec((1, D, 2 * F),
                         lambda g, d, a, wa, wb, p: (wb[g, 0], 0, 0)),
            pl.BlockSpec((1, F, D),
                         lambda g, d, a, wa, wb, p: (d[g, 0], 0, 0)),
        ],
        out_specs=pl.BlockSpec((BLK, D), lambda g, d, a, wa, wb, p: (g, 0)),
        scratch_shapes=[
            pltpu.VMEM((D, 2 * F), jnp.bfloat16),
            pltpu.VMEM((F, D), jnp.bfloat16),
        ],
    )

    def body(d_ref, a_ref, wa_ref, wb_ref, p_ref, xs_ref, gs_ref, wga_ref,
             wgb_ref, wd_ref, y_ref, wgu_bf, wd_bf):
        return _gemm_body(d_ref, a_ref, p_ref, xs_ref, gs_ref, wga_ref,
                          wgb_ref, wd_ref, y_ref, wgu_bf, wd_bf)

    return pl.pallas_call(
        body,
        grid_spec=grid_spec,
        out_shape=jax.ShapeDtypeStruct((P_MOE, D), jnp.float32),
    )(dma, act, wa, wb, par, xs, gs, w_gate_up, w_gate_up, w_down)


def _shared_body(x_ref, wgu_ref, wd_ref, y_ref, wgu_bf, wd_bf):
    @pl.when(pl.program_id(0) == 0)
    def _():
        wgu_bf[...] = wgu_ref[...].astype(jnp.bfloat16)
        wd_bf[...] = wd_ref[...].astype(jnp.bfloat16)

    x = x_ref[...].astype(jnp.bfloat16)
    h = jnp.dot(x, wgu_bf[...], preferred_element_type=jnp.float32)
    gate = h[:, :S]
    up = h[:, S:]
    act = (gate * (1.0 / (1.0 + jnp.exp(-gate))) * up).astype(jnp.bfloat16)
    y_ref[...] = jnp.dot(act, wd_bf[...], preferred_element_type=jnp.float32)


def _shared_call(x, ws_gate_up, ws_down):
    return pl.pallas_call(
        _shared_body,
        grid=(T // BLK,),
        in_specs=[
            pl.BlockSpec((BLK, D), lambda i: (i, 0)),
            pl.BlockSpec((D, 2 * S), lambda i: (0, 0)),
            pl.BlockSpec((S, D), lambda i: (0, 0)),
        ],
        out_specs=pl.BlockSpec((BLK, D), lambda i: (i, 0)),
        out_shape=jax.ShapeDtypeStruct((T, D), jnp.float32),
        scratch_shapes=[
            pltpu.VMEM((D, 2 * S), jnp.bfloat16),
            pltpu.VMEM((S, D), jnp.bfloat16),
        ],
    )(x, ws_gate_up, ws_down)


@functools.cache
def _make_combine():
    mesh = plsc.VectorSubcoreMesh(core_axis_name="c", subcore_axis_name="s")
    n_sub = CHUNK // SUB
    n_col = D // LANES

    @functools.partial(
        pl.kernel,
        out_type=jax.ShapeDtypeStruct((T, D), jnp.float32),
        mesh=mesh,
        scratch_types=[
            [pltpu.VMEM((SUB,), jnp.int32) for _ in range(2)],
            [pltpu.VMEM((SUB,), jnp.int32) for _ in range(2)],
            [pltpu.VMEM((SUB, D), jnp.float32) for _ in range(2)],
            [pltpu.VMEM((SUB, D), jnp.float32) for _ in range(2)],
            [pltpu.VMEM((SUB, D), jnp.float32) for _ in range(2)],
            [pltpu.SemaphoreType.DMA for _ in range(2)],
        ],
    )
    def combine_k(y_hbm, sh_hbm, pos0_hbm, pos1_hbm,
                  out_hbm, idx0_v, idx1_v, r0_v, r1_v, acc_v, sems):
        wid = lax.axis_index("s") * NC + lax.axis_index("c")
        pending = [None, None]

        def start(slot, s):
            base = wid * CHUNK + s * SUB
            pltpu.sync_copy(pos0_hbm.at[pl.ds(base, SUB)], idx0_v[slot])
            pltpu.sync_copy(pos1_hbm.at[pl.ds(base, SUB)], idx1_v[slot])
            c0 = pltpu.async_copy(y_hbm.at[idx0_v[slot]], r0_v[slot],
                                  sems[slot])
            c1 = pltpu.async_copy(y_hbm.at[idx1_v[slot]], r1_v[slot],
                                  sems[slot])
            c2 = pltpu.async_copy(sh_hbm.at[pl.ds(base, SUB)], acc_v[slot],
                                  sems[slot])
            pending[slot] = (c0, c1, c2)

        start(0, 0)
        for s in range(n_sub):
            slot = s % 2
            if s + 1 < n_sub:
                start(1 - slot, s + 1)
            for c in pending[slot]:
                c.wait()
            acc = acc_v[slot]
            r0 = r0_v[slot]
            r1 = r1_v[slot]

            def row_body(r, _):
                def col_body(cc, _):
                    sl = pl.ds(cc * LANES, LANES)
                    plsc.addupdate(acc.at[r, sl], r0[r, sl] + r1[r, sl])
                    return 0

                lax.fori_loop(0, n_col, col_body, 0, unroll=8)
                return 0

            lax.fori_loop(0, SUB, row_body, 0)
            base = wid * CHUNK + s * SUB
            pltpu.sync_copy(acc, out_hbm.at[pl.ds(base, SUB)])

    return combine_k


def kernel(hidden_states, router_w, w_gate_up, w_down, ws_gate_up, ws_down):
    x = hidden_states
    logits = x @ router_w.T
    pos0, pos1, g0, g1, dma, act, wa, wb, par = _router_call(logits)
    pos0 = pos0.reshape(T)
    pos1 = pos1.reshape(T)

    xs, gs = _make_scatter()(x, pos0, pos1, g0, g1)
    sh = _shared_call(x, ws_gate_up, ws_down)
    y = _gemm_call(dma, act, wa, wb, par, xs, gs, w_gate_up, w_down)
    out = _make_combine()(y, sh, pos0, pos1)
    return out


# speedup vs baseline: 1.0202x; 1.0202x over previous
import functools

import jax
import jax.numpy as jnp
from jax import lax
from jax.experimental import pallas as pl
from jax.experimental.pallas import tpu as pltpu
from jax.experimental.pallas import tpu_sc as plsc

E = 8
D = 768
F = 768
S = 768
T = 2048

BLK = 512
G_MOE = 16
P_MOE = G_MOE * BLK

NC, NS = 2, 16
NW = NC * NS
CHUNK = T // NW
SUB = 16
LANES = 16
GW = 128


def _router_body(logits_ref, pos0_ref, pos1_ref, g0_ref, g1_ref,
                 dma_ref, act_ref):
    logits = logits_ref[...]
    col = lax.broadcasted_iota(jnp.int32, (T, E), 1)
    l0 = jnp.max(logits, axis=1, keepdims=True)
    i0 = jnp.min(jnp.where(logits == l0, col, E), axis=1, keepdims=True)
    masked = jnp.where(col == i0, -jnp.inf, logits)
    l1 = jnp.max(masked, axis=1, keepdims=True)
    i1 = jnp.min(jnp.where(masked == l1, col, E), axis=1, keepdims=True)
    ed = jnp.exp(l1 - l0)
    g0 = 1.0 / (1.0 + ed)
    g1 = ed / (1.0 + ed)
    g0_ref[...] = jnp.broadcast_to(g0, (T, GW))
    g1_ref[...] = jnp.broadcast_to(g1, (T, GW))

    m = ((col == i0) | (col == i1)).astype(jnp.float32)
    c = m
    sh = 1
    while sh < T:
        c = c + jnp.concatenate(
            [jnp.zeros((sh, E), jnp.float32), c[: T - sh]], axis=0)
        sh *= 2
    counts = c[T - 1: T, :]
    padded = jnp.ceil(counts * (1.0 / BLK)) * BLK
    off = padded
    sh = 1
    while sh < E:
        off = off + jnp.concatenate(
            [jnp.zeros((1, sh), jnp.float32), off[:, : E - sh]], axis=1)
        sh *= 2
    off = off - padded

    pos = off + c - 1.0
    pos0 = jnp.sum(jnp.where(col == i0, pos, 0.0), axis=1, keepdims=True)
    pos1 = jnp.sum(jnp.where(col == i1, pos, 0.0), axis=1, keepdims=True)
    pos0_ref[...] = pos0.astype(jnp.int32).reshape(16, 128)
    pos1_ref[...] = pos1.astype(jnp.int32).reshape(16, 128)

    nbb = off * (1.0 / BLK)
    used = jnp.sum(padded) * (1.0 / BLK)
    e_last = jnp.max(jnp.where(counts > 0.0,
                               col[:1, :].astype(jnp.float32), 0.0))
    gidx = lax.broadcasted_iota(jnp.int32, (G_MOE, E), 0).astype(jnp.float32)
    nbb_b = jnp.broadcast_to(nbb, (G_MOE, E))
    exp_g = jnp.sum((nbb_b <= gidx).astype(jnp.float32), axis=1,
                    keepdims=True) - 1.0
    gcol = lax.broadcasted_iota(jnp.int32, (G_MOE, 1), 0).astype(jnp.float32)
    active = gcol < used
    dma_ref[...] = jnp.where(active, exp_g, e_last).astype(jnp.int32)
    act_ref[...] = active.astype(jnp.int32)


def _router_call(logits):
    return pl.pallas_call(
        _router_body,
        out_shape=(
            jax.ShapeDtypeStruct((16, 128), jnp.int32),
            jax.ShapeDtypeStruct((16, 128), jnp.int32),
            jax.ShapeDtypeStruct((T, GW), jnp.float32),
            jax.ShapeDtypeStruct((T, GW), jnp.float32),
            jax.ShapeDtypeStruct((G_MOE, 1), jnp.int32),
            jax.ShapeDtypeStruct((G_MOE, 1), jnp.int32),
        ),
    )(logits)


@functools.cache
def _make_scatter():
    mesh = plsc.VectorSubcoreMesh(core_axis_name="c", subcore_axis_name="s")

    @functools.partial(
        pl.kernel,
        out_type=(
            jax.ShapeDtypeStruct((P_MOE, D), jnp.float32),
            jax.ShapeDtypeStruct((P_MOE, GW), jnp.float32),
        ),
        mesh=mesh,
        scratch_types=[
            pltpu.VMEM((CHUNK,), jnp.int32),
            pltpu.VMEM((CHUNK,), jnp.int32),
            pltpu.VMEM((CHUNK, D), jnp.float32),
            pltpu.VMEM((CHUNK, GW), jnp.float32),
            pltpu.VMEM((CHUNK, GW), jnp.float32),
            pltpu.SemaphoreType.DMA,
        ],
    )
    def scatter_k(x_hbm, pos0_hbm, pos1_hbm, g0_hbm, g1_hbm, xs_hbm, gs_hbm,
                  idx0_v, idx1_v, rows_v, gr0_v, gr1_v, sem):
        wid = lax.axis_index("s") * NC + lax.axis_index("c")
        base = wid * CHUNK
        pltpu.sync_copy(x_hbm.at[pl.ds(base, CHUNK)], rows_v)
        pltpu.sync_copy(pos0_hbm.at[pl.ds(base, CHUNK)], idx0_v)
        pltpu.sync_copy(pos1_hbm.at[pl.ds(base, CHUNK)], idx1_v)
        pltpu.sync_copy(g0_hbm.at[pl.ds(base, CHUNK)], gr0_v)
        pltpu.sync_copy(g1_hbm.at[pl.ds(base, CHUNK)], gr1_v)
        c0 = pltpu.async_copy(rows_v, xs_hbm.at[idx0_v], sem)
        c1 = pltpu.async_copy(rows_v, xs_hbm.at[idx1_v], sem)
        c2 = pltpu.async_copy(gr0_v, gs_hbm.at[idx0_v], sem)
        c3 = pltpu.async_copy(gr1_v, gs_hbm.at[idx1_v], sem)
        c0.wait()
        c1.wait()
        c2.wait()
        c3.wait()

    return scatter_k


def _gemm_body(dma_ref, act_ref, xs_ref, gs_ref, wgu_ref, wd_ref, y_ref,
               wgu_bf, wd_bf):
    g = pl.program_id(0)
    prev = dma_ref[jnp.maximum(g - 1, 0), 0]
    changed = jnp.logical_or(g == 0, dma_ref[g, 0] != prev)

    @pl.when(changed)
    def _():
        wgu_bf[...] = wgu_ref[0].astype(jnp.bfloat16)
        wd_bf[...] = wd_ref[0].astype(jnp.bfloat16)

    @pl.when(act_ref[g, 0] > 0)
    def _():
        x = xs_ref[...].astype(jnp.bfloat16)
        h = jnp.dot(x, wgu_bf[...], preferred_element_type=jnp.float32)
        gate = h[:, :F]
        up = h[:, F:]
        act = (gate * (1.0 / (1.0 + jnp.exp(-gate))) * up).astype(jnp.bfloat16)
        y = jnp.dot(act, wd_bf[...], preferred_element_type=jnp.float32)
        y_ref[...] = y * gs_ref[:, :1]


def _gemm_call(dma, act, xs, gs, w_gate_up, w_down):
    grid_spec = pltpu.PrefetchScalarGridSpec(
        num_scalar_prefetch=2,
        grid=(G_MOE,),
        in_specs=[
            pl.BlockSpec((BLK, D), lambda g, dma, act: (g, 0)),
            pl.BlockSpec((BLK, GW), lambda g, dma, act: (g, 0)),
            pl.BlockSpec((1, D, 2 * F),
                         lambda g, dma, act: (dma[g, 0], 0, 0)),
            pl.BlockSpec((1, F, D), lambda g, dma, act: (dma[g, 0], 0, 0)),
        ],
        out_specs=pl.BlockSpec((BLK, D), lambda g, dma, act: (g, 0)),
        scratch_shapes=[
            pltpu.VMEM((D, 2 * F), jnp.bfloat16),
            pltpu.VMEM((F, D), jnp.bfloat16),
        ],
    )
    return pl.pallas_call(
        _gemm_body,
        grid_spec=grid_spec,
        out_shape=jax.ShapeDtypeStruct((P_MOE, D), jnp.float32),
    )(dma, act, xs, gs, w_gate_up, w_down)


def _shared_body(x_ref, wgu_ref, wd_ref, y_ref, wgu_bf, wd_bf):
    @pl.when(pl.program_id(0) == 0)
    def _():
        wgu_bf[...] = wgu_ref[...].astype(jnp.bfloat16)
        wd_bf[...] = wd_ref[...].astype(jnp.bfloat16)

    x = x_ref[...].astype(jnp.bfloat16)
    h = jnp.dot(x, wgu_bf[...], preferred_element_type=jnp.float32)
    gate = h[:, :S]
    up = h[:, S:]
    act = (gate * (1.0 / (1.0 + jnp.exp(-gate))) * up).astype(jnp.bfloat16)
    y_ref[...] = jnp.dot(act, wd_bf[...], preferred_element_type=jnp.float32)


def _shared_call(x, ws_gate_up, ws_down):
    return pl.pallas_call(
        _shared_body,
        grid=(T // BLK,),
        in_specs=[
            pl.BlockSpec((BLK, D), lambda i: (i, 0)),
            pl.BlockSpec((D, 2 * S), lambda i: (0, 0)),
            pl.BlockSpec((S, D), lambda i: (0, 0)),
        ],
        out_specs=pl.BlockSpec((BLK, D), lambda i: (i, 0)),
        out_shape=jax.ShapeDtypeStruct((T, D), jnp.float32),
        scratch_shapes=[
            pltpu.VMEM((D, 2 * S), jnp.bfloat16),
            pltpu.VMEM((S, D), jnp.bfloat16),
        ],
    )(x, ws_gate_up, ws_down)


@functools.cache
def _make_combine():
    mesh = plsc.VectorSubcoreMesh(core_axis_name="c", subcore_axis_name="s")
    n_sub = CHUNK // SUB
    n_col = D // LANES

    @functools.partial(
        pl.kernel,
        out_type=jax.ShapeDtypeStruct((T, D), jnp.float32),
        mesh=mesh,
        scratch_types=[
            [pltpu.VMEM((SUB,), jnp.int32) for _ in range(2)],
            [pltpu.VMEM((SUB,), jnp.int32) for _ in range(2)],
            [pltpu.VMEM((SUB, D), jnp.float32) for _ in range(2)],
            [pltpu.VMEM((SUB, D), jnp.float32) for _ in range(2)],
            [pltpu.VMEM((SUB, D), jnp.float32) for _ in range(2)],
            [pltpu.SemaphoreType.DMA for _ in range(2)],
        ],
    )
    def combine_k(y_hbm, sh_hbm, pos0_hbm, pos1_hbm,
                  out_hbm, idx0_v, idx1_v, r0_v, r1_v, acc_v, sems):
        wid = lax.axis_index("s") * NC + lax.axis_index("c")
        pending = [None, None]

        def start(slot, s):
            base = wid * CHUNK + s * SUB
            pltpu.sync_copy(pos0_hbm.at[pl.ds(base, SUB)], idx0_v[slot])
            pltpu.sync_copy(pos1_hbm.at[pl.ds(base, SUB)], idx1_v[slot])
            c0 = pltpu.async_copy(y_hbm.at[idx0_v[slot]], r0_v[slot],
                                  sems[slot])
            c1 = pltpu.async_copy(y_hbm.at[idx1_v[slot]], r1_v[slot],
                                  sems[slot])
            c2 = pltpu.async_copy(sh_hbm.at[pl.ds(base, SUB)], acc_v[slot],
                                  sems[slot])
            pending[slot] = (c0, c1, c2)

        start(0, 0)
        for s in range(n_sub):
            slot = s % 2
            if s + 1 < n_sub:
                start(1 - slot, s + 1)
            for c in pending[slot]:
                c.wait()
            acc = acc_v[slot]
            r0 = r0_v[slot]
            r1 = r1_v[slot]

            def row_body(r, _):
                def col_body(cc, _):
                    sl = pl.ds(cc * LANES, LANES)
                    plsc.addupdate(acc.at[r, sl], r0[r, sl] + r1[r, sl])
                    return 0

                lax.fori_loop(0, n_col, col_body, 0, unroll=8)
                return 0

            lax.fori_loop(0, SUB, row_body, 0)
            base = wid * CHUNK + s * SUB
            pltpu.sync_copy(acc, out_hbm.at[pl.ds(base, SUB)])

    return combine_k


def kernel(hidden_states, router_w, w_gate_up, w_down, ws_gate_up, ws_down):
    x = hidden_states
    logits = x @ router_w.T
    pos0, pos1, g0, g1, dma, act = _router_call(logits)
    pos0 = pos0.reshape(T)
    pos1 = pos1.reshape(T)

    xs, gs = _make_scatter()(x, pos0, pos1, g0, g1)
    sh = _shared_call(x, ws_gate_up, ws_down)
    y = _gemm_call(dma, act, xs, gs, w_gate_up, w_down)
    out = _make_combine()(y, sh, pos0, pos1)
    return out
